# baseline (speedup 1.0000x reference)
import functools

import jax
import jax.numpy as jnp
from jax import lax
from jax.experimental import pallas as pl
from jax.experimental.pallas import tpu as pltpu
from jax.experimental.pallas import tpu_sc as plsc

B, T, D, H = 16, 2048, 128, 64
L = 16


def _sc_body(ta_hbm, tb_hbm, la_hbm, lb_hbm, fa_hbm, fb_hbm, gf_hbm,
             tba, tbb, la_v, lb_v, rows_a, rows_b, gsel,
             sem_ta, sem_tb, sem_la, sem_lb, sem_a, sem_b):
    row = lax.axis_index("s")

    cp_ta = pltpu.async_copy(ta_hbm.at[pl.ds(row * T, T)], tba, sem_ta)
    cp_tb = pltpu.async_copy(tb_hbm.at[pl.ds(row * T, T)], tbb, sem_tb)
    cp_la = pltpu.async_copy(la_hbm, la_v, sem_la)
    cp_lb = pltpu.async_copy(lb_hbm, lb_v, sem_lb)
    cp_la.wait()
    cp_lb.wait()

    lane = lax.iota(jnp.int32, L)
    lva = la_v[...]
    lvb = lb_v[...]
    len_a_s = lva[0]
    len_b_s = lvb[0]
    for i in range(1, L):
        len_a_s = jnp.where(row == i, lva[i], len_a_s)
        len_b_s = jnp.where(row == i, lvb[i], len_b_s)
    len_a = jnp.broadcast_to(len_a_s, (L,))
    len_b = jnp.broadcast_to(len_b_s, (L,))
    cp_ta.wait()
    cp_tb.wait()

    U = 4
    STEP = U * L

    def body(c, carry):
        base = c * STEP
        out = []
        for u in range(2 * U):
            vmax, vidx = carry[u]
            pos = base + (u % U) * L + lane
            src = tba if u < U else tbb
            v = src[pl.ds(base + (u % U) * L, L)]
            val = jnp.where(pos < (len_a if u < U else len_b), v, 0.0)
            take = val >= vmax
            cidx = pos if u < U else pos + T
            out.append((jnp.where(take, val, vmax),
                        jnp.where(take, cidx, vidx)))
        return tuple(out)

    init = tuple((jnp.zeros((L,), jnp.float32), jnp.zeros((L,), jnp.int32))
                 for _ in range(2 * U))
    streams = lax.fori_loop(0, T // STEP, body, init)

    sub = list(streams)
    while len(sub) > 1:
        nxt = []
        for a, b2 in zip(sub[0::2], sub[1::2]):
            (va, ia), (vb, ib) = a, b2
            takeb = (vb > va) | ((vb == va) & (ib > ia))
            nxt.append((jnp.where(takeb, vb, va),
                        jnp.where(takeb, ib, ia)))
        sub = nxt
    vmax, vidx = sub[0]

    mxs = [vmax[i] for i in range(L)]
    ixs = [vidx[i] for i in range(L)]
    m = mxs[0]
    for s in mxs[1:]:
        m = jnp.maximum(m, s)
    pc = jnp.int32(-1)
    for s, ix in zip(mxs, ixs):
        pc = jnp.where(s == m, jnp.maximum(pc, ix), pc)

    sel_b = pc >= T
    p = jnp.where(sel_b, pc - T, pc)

    idxs = jnp.broadcast_to(row.astype(jnp.int32) * T + p, (L,))
    cp_a = pltpu.async_copy(fa_hbm.at[idxs], rows_a, sem_a)
    cp_b = pltpu.async_copy(fb_hbm.at[idxs], rows_b, sem_b)
    cp_a.wait()
    cp_b.wait()
    for k in range(D // L):
        sl = pl.ds(k * L, L)
        gsel[sl] = jnp.where(sel_b, rows_b[0, sl], rows_a[0, sl])
    gsel[pl.ds(D, L)] = jnp.broadcast_to(
        jnp.where(sel_b, 1.0, 0.0).astype(jnp.float32), (L,))
    pltpu.sync_copy(gsel, gf_hbm.at[row])


@jax.jit
def _sc_call(time_a, time_b, len_a, len_b, fa_flat, fb_flat):
    mesh = plsc.VectorSubcoreMesh(core_axis_name="c", subcore_axis_name="s",
                                  num_cores=1)
    return pl.kernel(
        _sc_body,
        out_type=jax.ShapeDtypeStruct((B, D + L), jnp.float32),
        mesh=mesh,
        scratch_types=[
            pltpu.VMEM((T,), jnp.float32),
            pltpu.VMEM((T,), jnp.float32),
            pltpu.VMEM((L,), jnp.int32),
            pltpu.VMEM((L,), jnp.int32),
            pltpu.VMEM((L, D), jnp.float32),
            pltpu.VMEM((L, D), jnp.float32),
            pltpu.VMEM((D + L,), jnp.float32),
            pltpu.SemaphoreType.DMA,
            pltpu.SemaphoreType.DMA,
            pltpu.SemaphoreType.DMA,
            pltpu.SemaphoreType.DMA,
            pltpu.SemaphoreType.DMA,
            pltpu.SemaphoreType.DMA,
        ],
    )(time_a, time_b, len_a, len_b, fa_flat, fb_flat)


def _tc_body(gf_ref, wa_ref, ba_ref, wb_ref, bb_ref, ws_ref, bs_ref, out_ref):
    g = gf_ref[:, 0:D]
    flag_b = gf_ref[:, D:D + 1] > 0.5
    ha = jnp.dot(g, wa_ref[...], preferred_element_type=jnp.float32) + ba_ref[...]
    hb = jnp.dot(g, wb_ref[...], preferred_element_type=jnp.float32) + bb_ref[...]
    h = jnp.where(flag_b, hb, ha)
    out_ref[...] = (
        jnp.dot(h, ws_ref[...], preferred_element_type=jnp.float32) + bs_ref[...])


@jax.jit
def _tc_call(gf, W_a, b_a, W_b, b_b, W_seq, b_seq):
    return pl.pallas_call(
        _tc_body,
        out_shape=jax.ShapeDtypeStruct((B, H), jnp.float32),
    )(gf, W_a, b_a, W_b, b_b, W_seq, b_seq)


def kernel(time_a, feat_a, len_a, time_b, feat_b, len_b,
           W_a, b_a, W_b, b_b, W_seq, b_seq):
    gf = _sc_call(time_a.reshape(B * T), time_b.reshape(B * T),
                  len_a.astype(jnp.int32), len_b.astype(jnp.int32),
                  feat_a.reshape(B * T, D), feat_b.reshape(B * T, D))
    return _tc_call(gf, W_a, b_a.reshape(1, H), W_b, b_b.reshape(1, H),
                    W_seq, b_seq.reshape(1, H))


# speedup vs baseline: 1.0205x; 1.0205x over previous
import jax
import jax.numpy as jnp
from jax import lax
from jax.experimental import pallas as pl
from jax.experimental.pallas import tpu as pltpu
from jax.experimental.pallas import tpu_sc as plsc

B, T, D, H = 16, 2048, 128, 64
L = 16


def _sc_body(ta_hbm, tb_hbm, lens_hbm, fa_hbm, fb_hbm, gf_hbm,
             tba, tbb, lens_v, rows_a, rows_b, gsel,
             sem_ta, sem_tb, sem_ln, sem_a, sem_b):
    row = lax.axis_index("s")

    cp_ta = pltpu.async_copy(ta_hbm.at[row], tba, sem_ta)
    cp_tb = pltpu.async_copy(tb_hbm.at[row], tbb, sem_tb)
    cp_ln = pltpu.async_copy(lens_hbm.at[row], lens_v, sem_ln)
    cp_ta.wait()
    cp_tb.wait()
    cp_ln.wait()

    lane = lax.iota(jnp.int32, L)
    len_a = lens_v[pl.ds(0, L)].astype(jnp.int32)
    len_b = lens_v[pl.ds(L, L)].astype(jnp.int32)

    U = 4
    STEP = U * L

    def body(c, carry):
        base = c * STEP
        out = []
        for u in range(2 * U):
            vmax, vidx = carry[u]
            pos = base + (u % U) * L + lane
            src = tba if u < U else tbb
            v = src[pl.ds(base + (u % U) * L, L)]
            val = jnp.where(pos < (len_a if u < U else len_b), v, 0.0)
            take = val >= vmax
            out.append((jnp.where(take, val, vmax),
                        jnp.where(take, pos, vidx)))
        return tuple(out)

    init = tuple((jnp.zeros((L,), jnp.float32), jnp.zeros((L,), jnp.int32))
                 for _ in range(2 * U))
    streams = lax.fori_loop(0, T // STEP, body, init)

    def merge(sub):
        sub = list(sub)
        while len(sub) > 1:
            nxt = []
            for a, b2 in zip(sub[0::2], sub[1::2]):
                (va, ia), (vb, ib) = a, b2
                takeb = (vb > va) | ((vb == va) & (ib > ia))
                nxt.append((jnp.where(takeb, vb, va),
                            jnp.where(takeb, ib, ia)))
            sub = nxt
        return sub[0]

    def reduce_lanes(vmax, vidx):
        mxs = [vmax[i] for i in range(L)]
        ixs = [vidx[i] for i in range(L)]
        m = mxs[0]
        for s in mxs[1:]:
            m = jnp.maximum(m, s)
        p = jnp.int32(-1)
        for s, ix in zip(mxs, ixs):
            p = jnp.where(s == m, jnp.maximum(p, ix), p)
        return m, p

    m_a, p_a = reduce_lanes(*merge(streams[:U]))
    m_b, p_b = reduce_lanes(*merge(streams[U:]))

    sel_b = m_b >= m_a
    p = jnp.where(sel_b, p_b, p_a)

    idxs = jnp.broadcast_to(row.astype(jnp.int32) * T + p, (L,))
    cp_a = pltpu.async_copy(fa_hbm.at[idxs], rows_a, sem_a)
    cp_b = pltpu.async_copy(fb_hbm.at[idxs], rows_b, sem_b)
    cp_a.wait()
    cp_b.wait()
    for k in range(D // L):
        sl = pl.ds(k * L, L)
        gsel[sl] = jnp.where(sel_b, rows_b[0, sl], rows_a[0, sl])
    gsel[pl.ds(D, L)] = jnp.broadcast_to(
        jnp.where(sel_b, 1.0, 0.0).astype(jnp.float32), (L,))
    pltpu.sync_copy(gsel, gf_hbm.at[row])


@jax.jit
def _sc_call(time_a, time_b, lens2, fa_flat, fb_flat):
    mesh = plsc.VectorSubcoreMesh(core_axis_name="c", subcore_axis_name="s",
                                  num_cores=1)
    return pl.kernel(
        _sc_body,
        out_type=jax.ShapeDtypeStruct((B, D + L), jnp.float32),
        mesh=mesh,
        scratch_types=[
            pltpu.VMEM((T,), jnp.float32),
            pltpu.VMEM((T,), jnp.float32),
            pltpu.VMEM((2 * L,), jnp.float32),
            pltpu.VMEM((L, D), jnp.float32),
            pltpu.VMEM((L, D), jnp.float32),
            pltpu.VMEM((D + L,), jnp.float32),
            pltpu.SemaphoreType.DMA,
            pltpu.SemaphoreType.DMA,
            pltpu.SemaphoreType.DMA,
            pltpu.SemaphoreType.DMA,
            pltpu.SemaphoreType.DMA,
        ],
    )(time_a, time_b, lens2, fa_flat, fb_flat)


def _tc_body(gf_ref, wa_ref, ba_ref, wb_ref, bb_ref, ws_ref, bs_ref, out_ref):
    g = gf_ref[:, 0:D]
    flag_b = gf_ref[:, D:D + 1] > 0.5
    ha = jnp.dot(g, wa_ref[...], preferred_element_type=jnp.float32) + ba_ref[...]
    hb = jnp.dot(g, wb_ref[...], preferred_element_type=jnp.float32) + bb_ref[...]
    h = jnp.where(flag_b, hb, ha)
    out_ref[...] = (
        jnp.dot(h, ws_ref[...], preferred_element_type=jnp.float32) + bs_ref[...])


@jax.jit
def _tc_call(gf, W_a, b_a, W_b, b_b, W_seq, b_seq):
    return pl.pallas_call(
        _tc_body,
        out_shape=jax.ShapeDtypeStruct((B, H), jnp.float32),
    )(gf, W_a, b_a, W_b, b_b, W_seq, b_seq)


def kernel(time_a, feat_a, len_a, time_b, feat_b, len_b,
           W_a, b_a, W_b, b_b, W_seq, b_seq):
    la = jnp.broadcast_to(len_a.astype(jnp.float32)[:, None], (B, L))
    lb = jnp.broadcast_to(len_b.astype(jnp.float32)[:, None], (B, L))
    lens2 = jnp.concatenate([la, lb], axis=1)
    gf = _sc_call(time_a, time_b, lens2,
                  feat_a.reshape(B * T, D), feat_b.reshape(B * T, D))
    return _tc_call(gf, W_a, b_a.reshape(1, H), W_b, b_b.reshape(1, H),
                    W_seq, b_seq.reshape(1, H))
